# baseline (speedup 1.0000x reference)
import functools

import jax
import jax.numpy as jnp
from jax import lax
from jax.experimental import pallas as pl
from jax.experimental.pallas import tpu as pltpu
from jax.experimental.pallas import tpu_sc as plsc

N = 10000
E = 320000
C = 128
NC = 2
NS = 16
NW = NC * NS
EB = 128
NBLK = E // EB
BPT = 80
HB = 40
NPB = NW * BPT
ZR = 624
ZL = N - (NS - 1) * ZR

_mesh = plsc.VectorSubcoreMesh(
    core_axis_name="c", subcore_axis_name="s", num_cores=NC, num_subcores=NS
)


def _zero_acc(sid, zrow_hbm, acc_sh):
    @pl.when(sid < NS - 1)
    def _():
        pltpu.sync_copy(zrow_hbm.at[pl.ds(0, ZR)], acc_sh.at[pl.ds(sid * ZR, ZR)])

    @pl.when(sid == NS - 1)
    def _():
        pltpu.sync_copy(zrow_hbm, acc_sh.at[pl.ds((NS - 1) * ZR, ZL)])


def _writeback_acc(cid, sid, acc_sh, out0, out1):
    def _wb(r0, nr):
        @pl.when(cid == 0)
        def _():
            pltpu.sync_copy(acc_sh.at[pl.ds(r0, nr)], out0.at[pl.ds(r0, nr)])

        @pl.when(cid == 1)
        def _():
            pltpu.sync_copy(acc_sh.at[pl.ds(r0, nr)], out1.at[pl.ds(r0, nr)])

    @pl.when(sid < NS - 1)
    def _():
        _wb(sid * ZR, ZR)

    @pl.when(sid == NS - 1)
    def _():
        _wb((NS - 1) * ZR, ZL)


def _agg_loop(wid, tbl_hbm, acc_sh, src_hbm, dst_hbm, sidx, didx,
              buf0, buf1, sems, deg=None, first_half_hook=None):
    semg0, semg1, sems0, sems1 = sems
    degin_sh = degout_sh = ones_v = None
    if deg is not None:
        degin_sh, degout_sh, ones_v = deg

    def _half(h, hook):
        def valid(r):
            return (h * HB + r) * NW + wid < NBLK

        def fire_gather(r, buf, semg):
            @pl.when(valid(r))
            def _():
                pltpu.async_copy(tbl_hbm.at[sidx.at[r]], buf, semg)

        def wait_gather(r, buf, semg):
            @pl.when(valid(r))
            def _():
                pltpu.make_async_copy(tbl_hbm.at[sidx.at[r]], buf, semg).wait()

        def fire_scatter(r, buf, sems_):
            @pl.when(valid(r))
            def _():
                pltpu.async_copy(buf, acc_sh.at[didx.at[r]], sems_, add=True)
                if deg is not None:
                    pltpu.async_copy(ones_v, degin_sh.at[didx.at[r]], sems_,
                                     add=True)
                    pltpu.async_copy(ones_v, degout_sh.at[sidx.at[r]], sems_,
                                     add=True)

        def wait_scatter(r, buf, sems_):
            @pl.when(valid(r))
            def _():
                pltpu.make_async_copy(buf, acc_sh.at[didx.at[r]], sems_).wait()
                if deg is not None:
                    pltpu.make_async_copy(
                        ones_v, degin_sh.at[didx.at[r]], sems_).wait()
                    pltpu.make_async_copy(
                        ones_v, degout_sh.at[sidx.at[r]], sems_).wait()

        fire_gather(0, buf0, semg0)
        fire_gather(1, buf1, semg1)
        if hook is not None:
            hook()

        def body(p, carry):
            r0 = p * 2
            wait_gather(r0, buf0, semg0)
            fire_scatter(r0, buf0, sems0)
            wait_gather(r0 + 1, buf1, semg1)
            fire_scatter(r0 + 1, buf1, sems1)
            wait_scatter(r0, buf0, sems0)

            @pl.when(r0 + 2 < HB)
            def _():
                fire_gather(r0 + 2, buf0, semg0)

            wait_scatter(r0 + 1, buf1, sems1)

            @pl.when(r0 + 3 < HB)
            def _():
                fire_gather(r0 + 3, buf1, semg1)

            return carry

        lax.fori_loop(0, HB // 2, body, 0)

    for h in range(BPT // HB):
        pltpu.sync_copy(src_hbm.at[pl.ds(wid * BPT + h * HB, HB)], sidx)
        pltpu.sync_copy(dst_hbm.at[pl.ds(wid * BPT + h * HB, HB)], didx)
        _half(h, first_half_hook if h == 0 else None)


@functools.partial(
    pl.kernel,
    out_type=(
        jax.ShapeDtypeStruct((N, C), jnp.float32),
        jax.ShapeDtypeStruct((N, C), jnp.float32),
        jax.ShapeDtypeStruct((N,), jnp.float32),
        jax.ShapeDtypeStruct((N,), jnp.float32),
        jax.ShapeDtypeStruct((N,), jnp.float32),
        jax.ShapeDtypeStruct((N,), jnp.float32),
    ),
    mesh=_mesh,
    scratch_types=(
        pltpu.VMEM_SHARED((N, C), jnp.float32),
        pltpu.VMEM_SHARED((N,), jnp.float32),
        pltpu.VMEM_SHARED((N,), jnp.float32),
        pltpu.VMEM((HB, EB), jnp.int32),
        pltpu.VMEM((HB, EB), jnp.int32),
        pltpu.VMEM((EB, C), jnp.float32),
        pltpu.VMEM((EB, C), jnp.float32),
        pltpu.VMEM((EB,), jnp.float32),
        pltpu.SemaphoreType.DMA,
        pltpu.SemaphoreType.DMA,
        pltpu.SemaphoreType.DMA,
        pltpu.SemaphoreType.DMA,
    ),
)
def _agg_x_deg(x_hbm, src_hbm, dst_hbm, zrow_hbm, z1_hbm,
               xsum0_out, xsum1_out, din0_out, din1_out, dout0_out, dout1_out,
               acc_sh, degin_sh, degout_sh,
               sidx, didx, buf0, buf1, ones_v, semg0, semg1, sems0, sems1):
    cid = lax.axis_index("c")
    sid = lax.axis_index("s")
    wid = sid * NC + cid

    for i in range(EB // 16):
        ones_v[pl.ds(i * 16, 16)] = jnp.ones((16,), jnp.float32)

    def _init():
        _zero_acc(sid, zrow_hbm, acc_sh)

        @pl.when(sid == 0)
        def _():
            pltpu.sync_copy(z1_hbm, degin_sh)
            pltpu.sync_copy(z1_hbm, degout_sh)

        plsc.subcore_barrier()

    _agg_loop(wid, x_hbm, acc_sh, src_hbm, dst_hbm, sidx, didx,
              buf0, buf1, (semg0, semg1, sems0, sems1),
              deg=(degin_sh, degout_sh, ones_v), first_half_hook=_init)

    plsc.subcore_barrier()

    _writeback_acc(cid, sid, acc_sh, xsum0_out, xsum1_out)

    @pl.when(sid == 0)
    def _():
        @pl.when(cid == 0)
        def _():
            pltpu.sync_copy(degin_sh, din0_out)
            pltpu.sync_copy(degout_sh, dout0_out)

        @pl.when(cid == 1)
        def _():
            pltpu.sync_copy(degin_sh, din1_out)
            pltpu.sync_copy(degout_sh, dout1_out)


@functools.partial(
    pl.kernel,
    out_type=(
        jax.ShapeDtypeStruct((N, C), jnp.float32),
        jax.ShapeDtypeStruct((N, C), jnp.float32),
    ),
    mesh=_mesh,
    scratch_types=(
        pltpu.VMEM_SHARED((N, C), jnp.float32),
        pltpu.VMEM((HB, EB), jnp.int32),
        pltpu.VMEM((HB, EB), jnp.int32),
        pltpu.VMEM((EB, C), jnp.float32),
        pltpu.VMEM((EB, C), jnp.float32),
        pltpu.SemaphoreType.DMA,
        pltpu.SemaphoreType.DMA,
        pltpu.SemaphoreType.DMA,
        pltpu.SemaphoreType.DMA,
    ),
)
def _agg_y(y_hbm, src_hbm, dst_hbm, zrow_hbm,
           ysum0_out, ysum1_out, acc_sh,
           sidx, didx, buf0, buf1, semg0, semg1, sems0, sems1):
    cid = lax.axis_index("c")
    sid = lax.axis_index("s")
    wid = sid * NC + cid

    def _init():
        _zero_acc(sid, zrow_hbm, acc_sh)
        plsc.subcore_barrier()

    _agg_loop(wid, y_hbm, acc_sh, src_hbm, dst_hbm, sidx, didx,
              buf0, buf1, (semg0, semg1, sems0, sems1), first_half_hook=_init)

    plsc.subcore_barrier()

    _writeback_acc(cid, sid, acc_sh, ysum0_out, ysum1_out)


def _prep_body(x_ref, dinp_ref, doutp_ref, y_ref, rin_ref, iin_ref):
    din = jnp.maximum(jnp.sum(dinp_ref[...], axis=1, keepdims=True), 1.0)
    dout = jnp.maximum(jnp.sum(doutp_ref[...], axis=1, keepdims=True), 1.0)
    rin_ref[...] = lax.rsqrt(din)
    iin_ref[...] = 1.0 / din
    y_ref[...] = x_ref[...] * lax.rsqrt(dout)


def _combine_body(w_ref, xs0_ref, xs1_ref, ys0_ref, ys1_ref, x_ref, x0_ref,
                  rin_ref, iin_ref, wg_ref, wnb_ref, wself_ref, wgin_ref,
                  out_ref):
    w0 = w_ref[0]
    w1 = w_ref[1]
    w2 = w_ref[2]
    w3 = w_ref[3]
    xsum = xs0_ref[...] + xs1_ref[...]
    ysum = ys0_ref[...] + ys1_ref[...]
    gcn = ysum * rin_ref[...]
    mean = xsum * iin_ref[...]
    wx = wself_ref[...] * w1 + wgin_ref[...] * w2
    out_ref[...] = (
        jnp.dot(gcn, wg_ref[...], preferred_element_type=jnp.float32) * w0
        + jnp.dot(mean, wnb_ref[...], preferred_element_type=jnp.float32) * w1
        + jnp.dot(xsum, wgin_ref[...], preferred_element_type=jnp.float32) * w2
        + jnp.dot(x_ref[...], wx, preferred_element_type=jnp.float32)
        + (x_ref[...] + x0_ref[...]) * (0.5 * w3)
    )


RB = 1000


def _blockify(idx):
    pad = jnp.zeros((NPB * EB - E,), jnp.int32)
    return (
        jnp.concatenate([idx, pad])
        .reshape(BPT, NW, EB)
        .transpose(1, 0, 2)
        .reshape(NPB, EB)
    )


def kernel(x, edge_index, weights, x_0, W_gcn, W_sage_nb, W_sage_self, W_gin):
    src = _blockify(edge_index[0])
    dst = _blockify(edge_index[1])
    zrow = jnp.zeros((ZL, C), jnp.float32)
    z1 = jnp.zeros((N,), jnp.float32)

    xs0, xs1, din0, din1, dout0, dout1 = _agg_x_deg(x, src, dst, zrow, z1)

    dinT = jnp.stack([din0, din1], axis=1)
    doutT = jnp.stack([dout0, dout1], axis=1)
    y, rin, iin = pl.pallas_call(
        _prep_body,
        out_shape=(
            jax.ShapeDtypeStruct((N, C), jnp.float32),
            jax.ShapeDtypeStruct((N, 1), jnp.float32),
            jax.ShapeDtypeStruct((N, 1), jnp.float32),
        ),
    )(x, dinT, doutT)

    ys0, ys1 = _agg_y(y, src, dst, zrow)

    nsteps = N // RB
    row_block = pl.BlockSpec((RB, C), lambda i: (i, 0))
    col_block = pl.BlockSpec((RB, 1), lambda i: (i, 0))
    full_w = pl.BlockSpec((C, C), lambda i: (0, 0))
    out = pl.pallas_call(
        _combine_body,
        grid=(nsteps,),
        in_specs=[
            pl.BlockSpec(memory_space=pltpu.SMEM),
            row_block, row_block, row_block, row_block,
            row_block, row_block,
            col_block, col_block,
            full_w, full_w, full_w, full_w,
        ],
        out_specs=row_block,
        out_shape=jax.ShapeDtypeStruct((N, C), jnp.float32),
    )(
        weights,
        xs0, xs1, ys0, ys1,
        x, x_0, rin, iin,
        W_gcn, W_sage_nb, W_sage_self, W_gin,
    )
    return out


# speedup vs baseline: 1.2558x; 1.2558x over previous
import functools

import jax
import jax.numpy as jnp
from jax import lax
from jax.experimental import pallas as pl
from jax.experimental.pallas import tpu as pltpu
from jax.experimental.pallas import tpu_sc as plsc

N = 10000
E = 320000
C = 128
NC = 2
NS = 16
NW = NC * NS
EB = 128
NBLK = E // EB
BPT = 80
HB = 40
NPB = NW * BPT
ZR = 624
ZL = N - (NS - 1) * ZR

_mesh = plsc.VectorSubcoreMesh(
    core_axis_name="c", subcore_axis_name="s", num_cores=NC, num_subcores=NS
)


def _zero_acc(sid, zrow_hbm, acc_sh):
    @pl.when(sid < NS - 1)
    def _():
        pltpu.sync_copy(zrow_hbm.at[pl.ds(0, ZR)], acc_sh.at[pl.ds(sid * ZR, ZR)])

    @pl.when(sid == NS - 1)
    def _():
        pltpu.sync_copy(zrow_hbm, acc_sh.at[pl.ds((NS - 1) * ZR, ZL)])


def _writeback_acc(cid, sid, acc_sh, out0, out1):
    def _wb(r0, nr):
        @pl.when(cid == 0)
        def _():
            pltpu.sync_copy(acc_sh.at[pl.ds(r0, nr)], out0.at[pl.ds(r0, nr)])

        @pl.when(cid == 1)
        def _():
            pltpu.sync_copy(acc_sh.at[pl.ds(r0, nr)], out1.at[pl.ds(r0, nr)])

    @pl.when(sid < NS - 1)
    def _():
        _wb(sid * ZR, ZR)

    @pl.when(sid == NS - 1)
    def _():
        _wb((NS - 1) * ZR, ZL)


def _agg_loop(wid, tbl_hbm, acc_sh, src_hbm, dst_hbm, sidx, didx,
              buf0, buf1, sems, deg=None, first_half_hook=None):
    semg0, semg1 = sems[0], sems[1]

    def _half(h, hook):
        def valid(r):
            return (h * HB + r) * NW + wid < NBLK

        def fire(r, buf, sem):
            @pl.when(valid(r))
            def _():
                pltpu.async_copy(tbl_hbm.at[sidx.at[r]], buf, sem)

        def drain(r, buf, sem):
            @pl.when(valid(r))
            def _():
                pltpu.make_async_copy(tbl_hbm.at[sidx.at[r]], buf, sem).wait()
                pltpu.sync_copy(buf, acc_sh.at[didx.at[r]], add=True)
                if deg is not None:
                    degin_sh, degout_sh, ones_v = deg
                    pltpu.sync_copy(ones_v, degin_sh.at[didx.at[r]], add=True)
                    pltpu.sync_copy(ones_v, degout_sh.at[sidx.at[r]], add=True)

        fire(0, buf0, semg0)
        if hook is not None:
            hook()

        def body(p, carry):
            r0 = p * 2
            fire(r0 + 1, buf1, semg1)
            drain(r0, buf0, semg0)

            @pl.when(r0 + 2 < HB)
            def _():
                fire(r0 + 2, buf0, semg0)

            drain(r0 + 1, buf1, semg1)
            return carry

        lax.fori_loop(0, HB // 2, body, 0)

    for h in range(BPT // HB):
        pltpu.sync_copy(src_hbm.at[pl.ds(wid * BPT + h * HB, HB)], sidx)
        pltpu.sync_copy(dst_hbm.at[pl.ds(wid * BPT + h * HB, HB)], didx)
        _half(h, first_half_hook if h == 0 else None)


@functools.partial(
    pl.kernel,
    out_type=(
        jax.ShapeDtypeStruct((N, C), jnp.float32),
        jax.ShapeDtypeStruct((N, C), jnp.float32),
        jax.ShapeDtypeStruct((N,), jnp.float32),
        jax.ShapeDtypeStruct((N,), jnp.float32),
        jax.ShapeDtypeStruct((N,), jnp.float32),
        jax.ShapeDtypeStruct((N,), jnp.float32),
    ),
    mesh=_mesh,
    scratch_types=(
        pltpu.VMEM_SHARED((N, C), jnp.float32),
        pltpu.VMEM_SHARED((N,), jnp.float32),
        pltpu.VMEM_SHARED((N,), jnp.float32),
        pltpu.VMEM((HB, EB), jnp.int32),
        pltpu.VMEM((HB, EB), jnp.int32),
        pltpu.VMEM((EB, C), jnp.float32),
        pltpu.VMEM((EB, C), jnp.float32),
        pltpu.VMEM((EB,), jnp.float32),
        pltpu.SemaphoreType.DMA,
        pltpu.SemaphoreType.DMA,
        pltpu.SemaphoreType.DMA,
        pltpu.SemaphoreType.DMA,
    ),
)
def _agg_x_deg(x_hbm, src_hbm, dst_hbm, zrow_hbm, z1_hbm,
               xsum0_out, xsum1_out, din0_out, din1_out, dout0_out, dout1_out,
               acc_sh, degin_sh, degout_sh,
               sidx, didx, buf0, buf1, ones_v, semg0, semg1, sems0, sems1):
    cid = lax.axis_index("c")
    sid = lax.axis_index("s")
    wid = sid * NC + cid

    for i in range(EB // 16):
        ones_v[pl.ds(i * 16, 16)] = jnp.ones((16,), jnp.float32)

    def _init():
        _zero_acc(sid, zrow_hbm, acc_sh)

        @pl.when(sid == 0)
        def _():
            pltpu.sync_copy(z1_hbm, degin_sh)
            pltpu.sync_copy(z1_hbm, degout_sh)

        plsc.subcore_barrier()

    _agg_loop(wid, x_hbm, acc_sh, src_hbm, dst_hbm, sidx, didx,
              buf0, buf1, (semg0, semg1, sems0, sems1),
              deg=(degin_sh, degout_sh, ones_v), first_half_hook=_init)

    plsc.subcore_barrier()

    _writeback_acc(cid, sid, acc_sh, xsum0_out, xsum1_out)

    @pl.when(sid == 0)
    def _():
        @pl.when(cid == 0)
        def _():
            pltpu.sync_copy(degin_sh, din0_out)
            pltpu.sync_copy(degout_sh, dout0_out)

        @pl.when(cid == 1)
        def _():
            pltpu.sync_copy(degin_sh, din1_out)
            pltpu.sync_copy(degout_sh, dout1_out)


@functools.partial(
    pl.kernel,
    out_type=(
        jax.ShapeDtypeStruct((N, C), jnp.float32),
        jax.ShapeDtypeStruct((N, C), jnp.float32),
    ),
    mesh=_mesh,
    scratch_types=(
        pltpu.VMEM_SHARED((N, C), jnp.float32),
        pltpu.VMEM((HB, EB), jnp.int32),
        pltpu.VMEM((HB, EB), jnp.int32),
        pltpu.VMEM((EB, C), jnp.float32),
        pltpu.VMEM((EB, C), jnp.float32),
        pltpu.SemaphoreType.DMA,
        pltpu.SemaphoreType.DMA,
        pltpu.SemaphoreType.DMA,
        pltpu.SemaphoreType.DMA,
    ),
)
def _agg_y(y_hbm, src_hbm, dst_hbm, zrow_hbm,
           ysum0_out, ysum1_out, acc_sh,
           sidx, didx, buf0, buf1, semg0, semg1, sems0, sems1):
    cid = lax.axis_index("c")
    sid = lax.axis_index("s")
    wid = sid * NC + cid

    def _init():
        _zero_acc(sid, zrow_hbm, acc_sh)
        plsc.subcore_barrier()

    _agg_loop(wid, y_hbm, acc_sh, src_hbm, dst_hbm, sidx, didx,
              buf0, buf1, (semg0, semg1, sems0, sems1), first_half_hook=_init)

    plsc.subcore_barrier()

    _writeback_acc(cid, sid, acc_sh, ysum0_out, ysum1_out)


def _prep_body(x_ref, dinp_ref, doutp_ref, y_ref, rin_ref, iin_ref):
    din = jnp.maximum(jnp.sum(dinp_ref[...], axis=1, keepdims=True), 1.0)
    dout = jnp.maximum(jnp.sum(doutp_ref[...], axis=1, keepdims=True), 1.0)
    rin_ref[...] = lax.rsqrt(din)
    iin_ref[...] = 1.0 / din
    y_ref[...] = x_ref[...] * lax.rsqrt(dout)


def _base_body(w_ref, x_ref, x0_ref, wself_ref, wgin_ref, base_ref):
    w1 = w_ref[1]
    w2 = w_ref[2]
    w3 = w_ref[3]
    wx = wself_ref[...] * w1 + wgin_ref[...] * w2
    base_ref[...] = (
        jnp.dot(x_ref[...], wx, preferred_element_type=jnp.float32)
        + (x_ref[...] + x0_ref[...]) * (0.5 * w3)
    )


def _combine_body(w_ref, xs0_ref, xs1_ref, ys0_ref, ys1_ref, base_ref,
                  rin_ref, iin_ref, wg_ref, wnb_ref, wgin_ref, out_ref):
    w0 = w_ref[0]
    w1 = w_ref[1]
    w2 = w_ref[2]
    xsum = xs0_ref[...] + xs1_ref[...]
    ysum = ys0_ref[...] + ys1_ref[...]
    gcn = ysum * rin_ref[...]
    mean = xsum * iin_ref[...]
    out_ref[...] = (
        jnp.dot(gcn, wg_ref[...], preferred_element_type=jnp.float32) * w0
        + jnp.dot(mean, wnb_ref[...], preferred_element_type=jnp.float32) * w1
        + jnp.dot(xsum, wgin_ref[...], preferred_element_type=jnp.float32) * w2
        + base_ref[...]
    )


RB = 1000


def _blockify(idx):
    pad = jnp.zeros((NPB * EB - E,), jnp.int32)
    return (
        jnp.concatenate([idx, pad])
        .reshape(BPT, NW, EB)
        .transpose(1, 0, 2)
        .reshape(NPB, EB)
    )


def kernel(x, edge_index, weights, x_0, W_gcn, W_sage_nb, W_sage_self, W_gin):
    src = _blockify(edge_index[0])
    dst = _blockify(edge_index[1])
    zrow = jnp.zeros((ZL, C), jnp.float32)
    z1 = jnp.zeros((N,), jnp.float32)

    nsteps = N // RB
    row_block = pl.BlockSpec((RB, C), lambda i: (i, 0))
    col_block = pl.BlockSpec((RB, 1), lambda i: (i, 0))
    full_w = pl.BlockSpec((C, C), lambda i: (0, 0))

    base = pl.pallas_call(
        _base_body,
        grid=(nsteps,),
        in_specs=[
            pl.BlockSpec(memory_space=pltpu.SMEM),
            row_block, row_block, full_w, full_w,
        ],
        out_specs=row_block,
        out_shape=jax.ShapeDtypeStruct((N, C), jnp.float32),
    )(weights, x, x_0, W_sage_self, W_gin)

    xs0, xs1, din0, din1, dout0, dout1 = _agg_x_deg(x, src, dst, zrow, z1)

    dinT = jnp.stack([din0, din1], axis=1)
    doutT = jnp.stack([dout0, dout1], axis=1)
    y, rin, iin = pl.pallas_call(
        _prep_body,
        out_shape=(
            jax.ShapeDtypeStruct((N, C), jnp.float32),
            jax.ShapeDtypeStruct((N, 1), jnp.float32),
            jax.ShapeDtypeStruct((N, 1), jnp.float32),
        ),
    )(x, dinT, doutT)

    ys0, ys1 = _agg_y(y, src, dst, zrow)

    out = pl.pallas_call(
        _combine_body,
        grid=(nsteps,),
        in_specs=[
            pl.BlockSpec(memory_space=pltpu.SMEM),
            row_block, row_block, row_block, row_block,
            row_block,
            col_block, col_block,
            full_w, full_w, full_w,
        ],
        out_specs=row_block,
        out_shape=jax.ShapeDtypeStruct((N, C), jnp.float32),
    )(
        weights,
        xs0, xs1, ys0, ys1,
        base, rin, iin,
        W_gcn, W_sage_nb, W_gin,
    )
    return out
